# Initial kernel scaffold; baseline (speedup 1.0000x reference)
import jax
import jax.numpy as jnp
from jax.experimental import pallas as pl


def kernel(edge_index, x, W1, b1, W2, b2, Wp, bp):
    raise NotImplementedError("write your pallas kernel here")


# speedup vs baseline: 6.8281x; 6.8281x over previous
import functools

import jax
import jax.numpy as jnp
from jax import lax
from jax.experimental import pallas as pl
from jax.experimental.pallas import tpu as pltpu
from jax.experimental.pallas import tpu_sc as plsc

N = 10000
E = 160000
F = 256
HALF = 128
NACC = 10240
EPAD = 163840
CHUNKS = 80
ROWS_PER_TILE = NACC // 16


def _sc_deg_body(dst_hbm, out_hbm, dstv, degv):
    ci = lax.axis_index("c")
    si = lax.axis_index("s")
    wid = si * 2 + ci
    pltpu.sync_copy(dst_hbm.at[wid], dstv)

    def zero(i, _):
        degv[pl.ds(i * 16, 16)] = jnp.zeros((16,), jnp.float32)
        return 0

    lax.fori_loop(0, NACC // 16, zero, 0)

    ones = jnp.ones((16,), jnp.float32)

    def hist(m, _):
        idx = dstv[pl.ds(m * 16, 16)]
        plsc.addupdate_scatter(degv, [idx], ones)
        return 0

    lax.fori_loop(0, (EPAD // 32) // 16, hist, 0)
    pltpu.sync_copy(degv, out_hbm.at[wid])


def _sc_agg_body(h_hbm, src_hbm, dst_hbm, out_hbm, srcv, dstv, rows, acc):
    ci = lax.axis_index("c")
    si = lax.axis_index("s")
    r0 = si * ROWS_PER_TILE
    pltpu.sync_copy(
        h_hbm.at[pl.ds(ci * NACC + r0, ROWS_PER_TILE)],
        acc.at[pl.ds(r0, ROWS_PER_TILE)],
    )
    plsc.subcore_barrier()
    pltpu.sync_copy(src_hbm.at[ci, si], srcv)
    pltpu.sync_copy(dst_hbm.at[si], dstv)

    def step(j, _):
        pltpu.sync_copy(h_hbm.at[srcv.at[j]], rows)
        pltpu.sync_copy(rows, acc.at[dstv.at[j]], add=True)
        return 0

    lax.fori_loop(0, CHUNKS, step, 0)
    plsc.subcore_barrier()
    pltpu.sync_copy(
        acc.at[pl.ds(r0, ROWS_PER_TILE)],
        out_hbm.at[pl.ds(ci * NACC + r0, ROWS_PER_TILE)],
    )


def _make_sc_deg():
    mesh = plsc.VectorSubcoreMesh(core_axis_name="c", subcore_axis_name="s")
    return pl.kernel(
        _sc_deg_body,
        mesh=mesh,
        out_type=jax.ShapeDtypeStruct((32, NACC), jnp.float32),
        scratch_types=[
            pltpu.VMEM((EPAD // 32,), jnp.int32),
            pltpu.VMEM((NACC,), jnp.float32),
        ],
        compiler_params=pltpu.CompilerParams(needs_layout_passes=False),
    )


def _make_sc_agg():
    mesh = plsc.VectorSubcoreMesh(core_axis_name="c", subcore_axis_name="s")
    return pl.kernel(
        _sc_agg_body,
        mesh=mesh,
        out_type=jax.ShapeDtypeStruct((2 * NACC, HALF), jnp.float32),
        scratch_types=[
            pltpu.VMEM((CHUNKS, 128), jnp.int32),
            pltpu.VMEM((CHUNKS, 128), jnp.int32),
            pltpu.VMEM((128, HALF), jnp.float32),
            pltpu.VMEM_SHARED((NACC, HALF), jnp.float32),
        ],
        compiler_params=pltpu.CompilerParams(needs_layout_passes=False),
    )


def _dinv_of(deg_parts):
    deg = jnp.sum(deg_parts, axis=0) + 1.0
    return lax.rsqrt(deg)


def _tc1_body(x_ref, deg_ref, w1_ref, o0_ref, o1_ref):
    dinv = _dinv_of(deg_ref[...])
    h = jnp.dot(x_ref[...], w1_ref[...], preferred_element_type=jnp.float32)
    hp = h * dinv[:, None]
    o0_ref[...] = hp[:, :HALF]
    o1_ref[...] = hp[:, HALF:]


def _tc2_body(s0_ref, s1_ref, deg_ref, b1_ref, w2_ref, o0_ref, o1_ref):
    dinv = _dinv_of(deg_ref[...])
    s = jnp.concatenate([s0_ref[...], s1_ref[...]], axis=1)
    h1 = jnp.maximum(s * dinv[:, None] + b1_ref[...], 0.0)
    h2 = jnp.dot(h1, w2_ref[...], preferred_element_type=jnp.float32)
    hp = h2 * dinv[:, None]
    o0_ref[...] = hp[:, :HALF]
    o1_ref[...] = hp[:, HALF:]


def _tc3_body(s0_ref, s1_ref, deg_ref, b2_ref, wp_ref, bp_ref, o_ref):
    dinv = _dinv_of(deg_ref[...])
    s = jnp.concatenate([s0_ref[...], s1_ref[...]], axis=1)
    h2 = s * dinv[:, None] + b2_ref[...]
    logit = jnp.dot(h2, wp_ref[...], preferred_element_type=jnp.float32)
    o_ref[...] = jax.nn.sigmoid(logit + bp_ref[...])


_BN = 1280
_GRID = NACC // _BN


def _row_spec(w):
    return pl.BlockSpec((_BN, w), lambda i: (i, 0))


def _rep_spec(shape):
    return pl.BlockSpec(shape, lambda i: tuple(0 for _ in shape))


def _tc1(x_pad, deg_parts, W1):
    return pl.pallas_call(
        _tc1_body,
        grid=(_GRID,),
        in_specs=[
            _row_spec(F),
            pl.BlockSpec((32, _BN), lambda i: (0, i)),
            _rep_spec((F, F)),
        ],
        out_specs=[_row_spec(HALF), _row_spec(HALF)],
        out_shape=[
            jax.ShapeDtypeStruct((NACC, HALF), jnp.float32),
            jax.ShapeDtypeStruct((NACC, HALF), jnp.float32),
        ],
    )(x_pad, deg_parts, W1)


def _tc2(s0, s1, deg_parts, b1, W2):
    return pl.pallas_call(
        _tc2_body,
        grid=(_GRID,),
        in_specs=[
            _row_spec(HALF),
            _row_spec(HALF),
            pl.BlockSpec((32, _BN), lambda i: (0, i)),
            _rep_spec((1, F)),
            _rep_spec((F, F)),
        ],
        out_specs=[_row_spec(HALF), _row_spec(HALF)],
        out_shape=[
            jax.ShapeDtypeStruct((NACC, HALF), jnp.float32),
            jax.ShapeDtypeStruct((NACC, HALF), jnp.float32),
        ],
    )(s0, s1, deg_parts, b1, W2)


def _tc3(s0, s1, deg_parts, b2, Wp_pad, bp_pad):
    return pl.pallas_call(
        _tc3_body,
        grid=(_GRID,),
        in_specs=[
            _row_spec(HALF),
            _row_spec(HALF),
            pl.BlockSpec((32, _BN), lambda i: (0, i)),
            _rep_spec((1, F)),
            _rep_spec((F, HALF)),
            _rep_spec((1, HALF)),
        ],
        out_specs=_row_spec(HALF),
        out_shape=jax.ShapeDtypeStruct((NACC, HALF), jnp.float32),
    )(s0, s1, deg_parts, b2, Wp_pad, bp_pad)


@jax.jit
def kernel(edge_index, x, W1, b1, W2, b2, Wp, bp):
    src = edge_index[0].astype(jnp.int32)
    dst = edge_index[1].astype(jnp.int32)
    pad = EPAD - E
    src_p = jnp.concatenate([src, jnp.zeros((pad,), jnp.int32)])
    dst_p = jnp.concatenate([dst, jnp.full((pad,), NACC - 1, jnp.int32)])

    dst_flat = dst_p.reshape(32, EPAD // 32)
    src_rs = src_p.reshape(16, CHUNKS, 128)
    dst_rs = dst_p.reshape(16, CHUNKS, 128)
    src_off = jnp.stack([src_rs, src_rs + NACC])

    x_pad = jnp.pad(x, ((0, NACC - N), (0, 0)))
    b1r = b1.reshape(1, F)
    b2r = b2.reshape(1, F)
    Wp_pad = jnp.pad(Wp, ((0, 0), (0, HALF - Wp.shape[1])))
    bp_pad = jnp.broadcast_to(bp.reshape(1, 1), (1, HALF))

    deg_parts = _make_sc_deg()(dst_flat)

    o0, o1 = _tc1(x_pad, deg_parts, W1)
    h1 = jnp.concatenate([o0, o1], axis=0)
    s1 = _make_sc_agg()(h1, src_off, dst_rs)

    o0, o1 = _tc2(s1[:NACC], s1[NACC:], deg_parts, b1r, W2)
    h2 = jnp.concatenate([o0, o1], axis=0)
    s2 = _make_sc_agg()(h2, src_off, dst_rs)

    out = _tc3(s2[:NACC], s2[NACC:], deg_parts, b2r, Wp_pad, bp_pad)
    return out[:N, :1]


# speedup vs baseline: 7.5747x; 1.1093x over previous
import functools

import jax
import jax.numpy as jnp
from jax import lax
from jax.experimental import pallas as pl
from jax.experimental.pallas import tpu as pltpu
from jax.experimental.pallas import tpu_sc as plsc

N = 10000
E = 160000
F = 256
HALF = 128
NACC = 10240
EPAD = 163840
CHUNKS = 80
CHUNK_W = 128
CHUNKS_H = CHUNKS // 2
ROWS_PER_TILE = NACC // 16


def _sc_deg_body(dst_hbm, out_hbm, dstv, degv):
    ci = lax.axis_index("c")
    si = lax.axis_index("s")
    wid = si * 2 + ci
    pltpu.sync_copy(dst_hbm.at[wid], dstv)

    def zero(i, _):
        degv[pl.ds(i * 16, 16)] = jnp.zeros((16,), jnp.float32)
        return 0

    lax.fori_loop(0, NACC // 16, zero, 0)

    ones = jnp.ones((16,), jnp.float32)

    def hist(m, _):
        idx = dstv[pl.ds(m * 16, 16)]
        plsc.addupdate_scatter(degv, [idx], ones)
        return 0

    lax.fori_loop(0, (EPAD // 32) // 16, hist, 0)
    pltpu.sync_copy(degv, out_hbm.at[wid])


def _sc_agg_body(h_hbm, src_hbm, dst_hbm, out_hbm, srcv, dstv, rows0, rows1,
                 acc, gsem0, gsem1, ssem0, ssem1):
    ci = lax.axis_index("c")
    si = lax.axis_index("s")
    r0 = si * ROWS_PER_TILE
    pltpu.sync_copy(
        h_hbm.at[pl.ds(ci * NACC + r0, ROWS_PER_TILE)],
        acc.at[pl.ds(r0, ROWS_PER_TILE)],
    )
    plsc.subcore_barrier()
    pltpu.sync_copy(dst_hbm.at[si], dstv)

    def gather(j, buf, sem):
        pltpu.async_copy(h_hbm.at[srcv.at[j]], buf, sem)

    def drain(buf, sem):
        pltpu.make_async_copy(h_hbm.at[pl.ds(0, CHUNK_W)], buf, sem).wait()

    def scat(j, buf, sem):
        pltpu.async_copy(buf, acc.at[dstv.at[j]], sem, add=True)

    for h0 in range(2):
        pltpu.sync_copy(src_hbm.at[ci, si, h0], srcv)
        base = h0 * CHUNKS_H
        gather(0, rows0, gsem0)
        gather(1, rows1, gsem1)

        def step(g, _):
            j0 = 2 * g
            j1 = j0 + 1
            drain(rows0, gsem0)
            scat(base + j0, rows0, ssem0)
            drain(rows1, gsem1)
            scat(base + j1, rows1, ssem1)
            drain(rows0, ssem0)

            @pl.when(j0 + 2 < CHUNKS_H)
            def _g0():
                gather(j0 + 2, rows0, gsem0)

            drain(rows1, ssem1)

            @pl.when(j1 + 2 < CHUNKS_H)
            def _g1():
                gather(j1 + 2, rows1, gsem1)

            return 0

        lax.fori_loop(0, CHUNKS_H // 2, step, 0)

    plsc.subcore_barrier()
    pltpu.sync_copy(
        acc.at[pl.ds(r0, ROWS_PER_TILE)],
        out_hbm.at[pl.ds(ci * NACC + r0, ROWS_PER_TILE)],
    )


def _make_sc_deg():
    mesh = plsc.VectorSubcoreMesh(core_axis_name="c", subcore_axis_name="s")
    return pl.kernel(
        _sc_deg_body,
        mesh=mesh,
        out_type=jax.ShapeDtypeStruct((32, NACC), jnp.float32),
        scratch_types=[
            pltpu.VMEM((EPAD // 32,), jnp.int32),
            pltpu.VMEM((NACC,), jnp.float32),
        ],
        compiler_params=pltpu.CompilerParams(needs_layout_passes=False),
    )


def _make_sc_agg():
    mesh = plsc.VectorSubcoreMesh(core_axis_name="c", subcore_axis_name="s")
    return pl.kernel(
        _sc_agg_body,
        mesh=mesh,
        out_type=jax.ShapeDtypeStruct((2 * NACC, HALF), jnp.float32),
        scratch_types=[
            pltpu.VMEM((CHUNKS_H, CHUNK_W), jnp.int32),
            pltpu.VMEM((CHUNKS, CHUNK_W), jnp.int32),
            pltpu.VMEM((CHUNK_W, HALF), jnp.float32),
            pltpu.VMEM((CHUNK_W, HALF), jnp.float32),
            pltpu.VMEM_SHARED((NACC, HALF), jnp.float32),
            pltpu.SemaphoreType.DMA,
            pltpu.SemaphoreType.DMA,
            pltpu.SemaphoreType.DMA,
            pltpu.SemaphoreType.DMA,
        ],
        compiler_params=pltpu.CompilerParams(needs_layout_passes=False),
    )


def _dinv_of(deg_parts):
    deg = jnp.sum(deg_parts, axis=0) + 1.0
    return lax.rsqrt(deg)


def _tc1_body(x_ref, deg_ref, w1_ref, o0_ref, o1_ref):
    dinv = _dinv_of(deg_ref[...])
    h = jnp.dot(x_ref[...], w1_ref[...], preferred_element_type=jnp.float32)
    hp = h * dinv[:, None]
    o0_ref[...] = hp[:, :HALF]
    o1_ref[...] = hp[:, HALF:]


def _tc2_body(s0_ref, s1_ref, deg_ref, b1_ref, w2_ref, o0_ref, o1_ref):
    dinv = _dinv_of(deg_ref[...])
    s = jnp.concatenate([s0_ref[...], s1_ref[...]], axis=1)
    h1 = jnp.maximum(s * dinv[:, None] + b1_ref[...], 0.0)
    h2 = jnp.dot(h1, w2_ref[...], preferred_element_type=jnp.float32)
    hp = h2 * dinv[:, None]
    o0_ref[...] = hp[:, :HALF]
    o1_ref[...] = hp[:, HALF:]


def _tc3_body(s0_ref, s1_ref, deg_ref, b2_ref, wp_ref, bp_ref, o_ref):
    dinv = _dinv_of(deg_ref[...])
    s = jnp.concatenate([s0_ref[...], s1_ref[...]], axis=1)
    h2 = s * dinv[:, None] + b2_ref[...]
    logit = jnp.dot(h2, wp_ref[...], preferred_element_type=jnp.float32)
    o_ref[...] = jax.nn.sigmoid(logit + bp_ref[...])


_BN = 1280
_GRID = NACC // _BN


def _row_spec(w):
    return pl.BlockSpec((_BN, w), lambda i: (i, 0))


def _rep_spec(shape):
    return pl.BlockSpec(shape, lambda i: tuple(0 for _ in shape))


def _tc1(x_pad, deg_parts, W1):
    return pl.pallas_call(
        _tc1_body,
        grid=(_GRID,),
        in_specs=[
            _row_spec(F),
            pl.BlockSpec((32, _BN), lambda i: (0, i)),
            _rep_spec((F, F)),
        ],
        out_specs=[_row_spec(HALF), _row_spec(HALF)],
        out_shape=[
            jax.ShapeDtypeStruct((NACC, HALF), jnp.float32),
            jax.ShapeDtypeStruct((NACC, HALF), jnp.float32),
        ],
    )(x_pad, deg_parts, W1)


def _tc2(s0, s1, deg_parts, b1, W2):
    return pl.pallas_call(
        _tc2_body,
        grid=(_GRID,),
        in_specs=[
            _row_spec(HALF),
            _row_spec(HALF),
            pl.BlockSpec((32, _BN), lambda i: (0, i)),
            _rep_spec((1, F)),
            _rep_spec((F, F)),
        ],
        out_specs=[_row_spec(HALF), _row_spec(HALF)],
        out_shape=[
            jax.ShapeDtypeStruct((NACC, HALF), jnp.float32),
            jax.ShapeDtypeStruct((NACC, HALF), jnp.float32),
        ],
    )(s0, s1, deg_parts, b1, W2)


def _tc3(s0, s1, deg_parts, b2, Wp_pad, bp_pad):
    return pl.pallas_call(
        _tc3_body,
        grid=(_GRID,),
        in_specs=[
            _row_spec(HALF),
            _row_spec(HALF),
            pl.BlockSpec((32, _BN), lambda i: (0, i)),
            _rep_spec((1, F)),
            _rep_spec((F, HALF)),
            _rep_spec((1, HALF)),
        ],
        out_specs=_row_spec(HALF),
        out_shape=jax.ShapeDtypeStruct((NACC, HALF), jnp.float32),
    )(s0, s1, deg_parts, b2, Wp_pad, bp_pad)


@jax.jit
def kernel(edge_index, x, W1, b1, W2, b2, Wp, bp):
    src = edge_index[0].astype(jnp.int32)
    dst = edge_index[1].astype(jnp.int32)
    pad = EPAD - E
    src_p = jnp.concatenate([src, jnp.zeros((pad,), jnp.int32)])
    dst_p = jnp.concatenate([dst, jnp.full((pad,), NACC - 1, jnp.int32)])

    dst_flat = dst_p.reshape(32, EPAD // 32)
    src_rs = src_p.reshape(16, 2, CHUNKS_H, CHUNK_W)
    dst_rs = dst_p.reshape(16, CHUNKS, CHUNK_W)
    src_off = jnp.stack([src_rs, src_rs + NACC])

    x_pad = jnp.pad(x, ((0, NACC - N), (0, 0)))
    b1r = b1.reshape(1, F)
    b2r = b2.reshape(1, F)
    Wp_pad = jnp.pad(Wp, ((0, 0), (0, HALF - Wp.shape[1])))
    bp_pad = jnp.broadcast_to(bp.reshape(1, 1), (1, HALF))

    deg_parts = _make_sc_deg()(dst_flat)

    o0, o1 = _tc1(x_pad, deg_parts, W1)
    h1 = jnp.concatenate([o0, o1], axis=0)
    s1 = _make_sc_agg()(h1, src_off, dst_rs)

    o0, o1 = _tc2(s1[:NACC], s1[NACC:], deg_parts, b1r, W2)
    h2 = jnp.concatenate([o0, o1], axis=0)
    s2 = _make_sc_agg()(h2, src_off, dst_rs)

    out = _tc3(s2[:NACC], s2[NACC:], deg_parts, b2r, Wp_pad, bp_pad)
    return out[:N, :1]
